# baseline (speedup 1.0000x reference)
import functools

import jax
import jax.numpy as jnp
from jax import lax
from jax.experimental import pallas as pl
from jax.experimental.pallas import tpu as pltpu
from jax.experimental.pallas import tpu_sc as plsc

N = 10000
E = 160000
D = 256
H = 256
C = 16
G = 64

NC = 2
NS = 16
HD = D // NC
K = 64
IB = 16
RPT = 640
TRASH = RPT
NPAD = NS * RPT
PW = 2560
NPH = 4
NG = NS * NPH
NH = 2
EH = E // NH
EPAD = E + NH * NG * 2 * K + IB * K


def _prep_edges(src, dst):
    tile = dst // RPT
    phase = jnp.minimum(src // PW, NPH - 1)
    group = (tile * NPH + phase).reshape(NH, EH)
    idx = jnp.arange(EH, dtype=jnp.int32)
    pks = jnp.sort(group * (1 << 18) + idx[None, :], axis=1)
    group_s = pks >> 18
    bounds = jax.vmap(
        lambda row: jnp.searchsorted(row, jnp.arange(NG + 1, dtype=jnp.int32))
    )(group_s).astype(jnp.int32)
    cnt = bounds[:, 1:] - bounds[:, :-1]
    padded = ((cnt + 2 * K - 1) // (2 * K)) * (2 * K)
    nstart = jnp.concatenate(
        [jnp.zeros((1,), jnp.int32), jnp.cumsum(padded.reshape(-1)).astype(jnp.int32)]
    )
    half_off = jnp.arange(NH, dtype=jnp.int32)[:, None] * NG
    fpos = (
        jnp.take(nstart[: NH * NG].reshape(NH, NG) - bounds[:, :NG], half_off + group_s)
        + idx[None, :]
    )
    perm = (pks & ((1 << 18) - 1)) + jnp.arange(NH, dtype=jnp.int32)[:, None] * EH
    sd_s = jnp.take(src * 16384 + dst, perm)
    src_s = sd_s >> 14
    dst_s = sd_s & 16383
    phase_s = group_s & (NPH - 1)
    srcl_s = src_s - jnp.minimum(phase_s * PW, N - PW)
    loc_s = dst_s - (group_s >> 2) * RPT
    packed = jnp.full((EPAD,), TRASH, jnp.int32).at[fpos.reshape(-1)].add(
        (srcl_s * 1024 + loc_s - TRASH).reshape(-1)
    )
    src_pad = packed >> 10
    dl_pad = packed & 1023
    meta = jnp.zeros((NS, 16), jnp.int32)
    for p in range(NPH):
        g = jnp.arange(NS, dtype=jnp.int32) * NPH + p
        for h in range(NH):
            st = jnp.take(nstart, h * NG + g)
            meta = meta.at[:, 4 * p + 2 * h].set(st)
            meta = meta.at[:, 4 * p + 2 * h + 1].set(st + jnp.take(padded[h], g))
    return src_pad, dl_pad, meta.reshape(-1)


def _sc_segment_sum(feat, src_pad, dl_pad, meta):
    mesh = plsc.VectorSubcoreMesh(core_axis_name="c", subcore_axis_name="s")

    @functools.partial(
        pl.kernel,
        out_type=jax.ShapeDtypeStruct((NPAD, D), jnp.float32),
        mesh=mesh,
        scratch_types=[
            pltpu.VMEM_SHARED((PW, HD), jnp.float32),
            pltpu.VMEM((IB * K,), jnp.int32),
            pltpu.VMEM((IB * K,), jnp.int32),
            pltpu.VMEM((K, HD), jnp.float32),
            pltpu.VMEM((K, HD), jnp.float32),
            pltpu.VMEM((RPT + 1, HD), jnp.float32),
            pltpu.VMEM((16,), jnp.int32),
            pltpu.SemaphoreType.DMA,
            pltpu.SemaphoreType.DMA,
        ],
    )
    def k(feat_h, src_h, dl_h, meta_h, out_h, fsh, sblk, dblk, buf_a, buf_b, acc, st, sem_a, sem_b):
        c = lax.axis_index("c")
        s = lax.axis_index("s")

        zero = jnp.zeros((16,), jnp.float32)

        def zrow(r, _):
            for q in range(HD // 16):
                acc[r, pl.ds(q * 16, 16)] = zero
            return 0

        lax.fori_loop(0, RPT + 1, zrow, 0)

        pltpu.sync_copy(meta_h.at[pl.ds(pl.multiple_of(s * 16, 8), 16)], st)
        sv = st[...]

        for p in range(NPH):
            plsc.subcore_barrier()

            @pl.when(s == 0)
            def _():
                pltpu.sync_copy(
                    feat_h.at[pl.ds(min(p * PW, N - PW), PW), pl.ds(c * HD, HD)],
                    fsh,
                )

            plsc.subcore_barrier()

            def run_range(s0, s1):
                nch = (s1 - s0) // K
                nblk = (nch + IB - 1) // IB

                def blk_body(blk, _):
                    base = pl.multiple_of(s0 + blk * (IB * K), 8)
                    pltpu.sync_copy(src_h.at[pl.ds(base, IB * K)], sblk)
                    pltpu.sync_copy(dl_h.at[pl.ds(base, IB * K)], dblk)
                    nc_here = jnp.minimum(IB, nch - blk * IB)
                    pltpu.async_copy(fsh.at[sblk.at[pl.ds(0, K)]], buf_a, sem_a)

                    def accum(bref, doff):
                        def gbody(g, _):
                            dv = dblk[pl.ds(doff + g * 16, 16)]
                            for j in range(16):
                                d = dv[j]
                                row = g * 16 + j
                                for q in range(HD // 16):
                                    plsc.addupdate(
                                        acc.at[d, pl.ds(q * 16, 16)],
                                        bref[row, pl.ds(q * 16, 16)],
                                    )
                            return 0

                        lax.fori_loop(0, K // 16, gbody, 0)

                    def pair(t, _):
                        c0 = t * 2
                        ob = pl.multiple_of((c0 + 1) * K, K)
                        cb = pltpu.async_copy(
                            fsh.at[sblk.at[pl.ds(ob, K)]], buf_b, sem_b
                        )
                        pltpu.make_async_copy(
                            feat_h.at[pl.ds(0, K), pl.ds(0, HD)], buf_a, sem_a
                        ).wait()
                        accum(buf_a, pl.multiple_of(c0 * K, K))

                        @pl.when(c0 + 2 < nc_here)
                        def _():
                            oa = pl.multiple_of((c0 + 2) * K, K)
                            pltpu.async_copy(fsh.at[sblk.at[pl.ds(oa, K)]], buf_a, sem_a)

                        cb.wait()
                        accum(buf_b, pl.multiple_of((c0 + 1) * K, K))
                        return 0

                    lax.fori_loop(0, nc_here // 2, pair, 0)
                    return 0

                lax.fori_loop(0, nblk, blk_body, 0)

            for h in range(NH):
                run_range(sv[4 * p + 2 * h], sv[4 * p + 2 * h + 1])

        pltpu.sync_copy(
            acc.at[pl.ds(0, RPT)],
            out_h.at[pl.ds(s * RPT, RPT), pl.ds(c * HD, HD)],
        )

    return k(feat, src_pad, dl_pad, meta)


def _root_tc(xin, w_root):
    bm = 2000
    grid = N // bm

    def body(x_ref, wr_ref, o_ref):
        o_ref[...] = jnp.dot(x_ref[...], wr_ref[...], preferred_element_type=jnp.float32)

    return pl.pallas_call(
        body,
        grid=(grid,),
        in_specs=[
            pl.BlockSpec((bm, D), lambda i: (i, 0)),
            pl.BlockSpec((D, H), lambda i: (0, 0)),
        ],
        out_specs=pl.BlockSpec((bm, H), lambda i: (i, 0)),
        out_shape=jax.ShapeDtypeStruct((N, H), jnp.float32),
    )(xin, w_root)


def _combine_tc(xr, agg, w_nbr, b):
    bm = 2000
    grid = N // bm

    def body(r_ref, a_ref, wn_ref, b_ref, o_ref):
        acc = r_ref[...] + jnp.dot(
            a_ref[...], wn_ref[...], preferred_element_type=jnp.float32
        )
        o_ref[...] = jnp.maximum(acc + b_ref[...], 0.0)

    return pl.pallas_call(
        body,
        grid=(grid,),
        in_specs=[
            pl.BlockSpec((bm, H), lambda i: (i, 0)),
            pl.BlockSpec((bm, D), lambda i: (i, 0)),
            pl.BlockSpec((D, H), lambda i: (0, 0)),
            pl.BlockSpec((1, H), lambda i: (0, 0)),
        ],
        out_specs=pl.BlockSpec((bm, H), lambda i: (i, 0)),
        out_shape=jax.ShapeDtypeStruct((N, H), jnp.float32),
    )(xr, agg, w_nbr, b.reshape(1, H))


def _pool_tc(h, batch, lin_w, lin_b):
    bm = 2000
    grid = N // bm
    batch3 = batch.reshape(grid, 1, bm)

    def body(h_ref, b_ref, w_ref, bias_ref, o_ref, acc, cnt):
        i = pl.program_id(0)

        @pl.when(i == 0)
        def _():
            acc[...] = jnp.zeros_like(acc)
            cnt[...] = jnp.zeros_like(cnt)

        bb = b_ref[0, :, :]
        gids = lax.broadcasted_iota(jnp.int32, (G, bm), 0)
        oh = (gids == bb).astype(jnp.float32)
        acc[...] += jnp.dot(oh, h_ref[...], preferred_element_type=jnp.float32)
        cnt[...] += jnp.sum(oh, axis=1, keepdims=True)

        @pl.when(i == grid - 1)
        def _():
            pooled = acc[...] / jnp.maximum(cnt[...], 1.0)
            o_ref[...] = (
                jnp.dot(pooled, w_ref[...], preferred_element_type=jnp.float32)
                + bias_ref[...]
            )

    return pl.pallas_call(
        body,
        grid=(grid,),
        in_specs=[
            pl.BlockSpec((bm, H), lambda i: (i, 0)),
            pl.BlockSpec((1, 1, bm), lambda i: (i, 0, 0)),
            pl.BlockSpec((H, C), lambda i: (0, 0)),
            pl.BlockSpec((1, C), lambda i: (0, 0)),
        ],
        out_specs=pl.BlockSpec((G, C), lambda i: (0, 0)),
        out_shape=jax.ShapeDtypeStruct((G, C), jnp.float32),
        scratch_shapes=[
            pltpu.VMEM((G, H), jnp.float32),
            pltpu.VMEM((G, 1), jnp.float32),
        ],
    )(h, batch3, lin_w, lin_b.reshape(1, C))


def kernel(x, edge_index, batch, W1_root, W1_nbr, b1, W2_root, W2_nbr, b2, lin_W, lin_b):
    src_pad, dl_pad, meta = _prep_edges(edge_index[0], edge_index[1])
    xr1 = _root_tc(x, W1_root)
    agg1 = _sc_segment_sum(x, src_pad, dl_pad, meta)[:N]
    h1 = _combine_tc(xr1, agg1, W1_nbr, b1)
    xr2 = _root_tc(h1, W2_root)
    agg2 = _sc_segment_sum(h1, src_pad, dl_pad, meta)[:N]
    h2 = _combine_tc(xr2, agg2, W2_nbr, b2)
    return _pool_tc(h2, batch, lin_W, lin_b)


# speedup vs baseline: 2.4169x; 2.4169x over previous
import functools

import jax
import jax.numpy as jnp
from jax import lax
from jax.experimental import pallas as pl
from jax.experimental.pallas import tpu as pltpu
from jax.experimental.pallas import tpu_sc as plsc

N = 10000
E = 160000
D = 256
H = 256
C = 16
G = 64

NC = 2
NS = 16
HD = D // NC
K = 64
IB = 16
RPT = 640
TRASH = RPT
NPAD = NS * RPT
PW = 2560
NPH = 4
NG = NS * NPH
EPAD = E + NG * 2 * K + IB * K


def _prep_edges(src, dst):
    tile = dst // RPT
    phase = jnp.minimum(src // PW, NPH - 1)
    group = tile * NPH + phase
    idx = jnp.arange(E, dtype=jnp.int32)
    (pks,) = lax.sort([group * (1 << 18) + idx], is_stable=False)
    group_s = pks >> 18
    bounds = jnp.searchsorted(group_s, jnp.arange(NG + 1, dtype=jnp.int32)).astype(
        jnp.int32
    )
    cnt = bounds[1:] - bounds[:-1]
    padded = ((cnt + 2 * K - 1) // (2 * K)) * (2 * K)
    nstart = jnp.concatenate(
        [jnp.zeros((1,), jnp.int32), jnp.cumsum(padded).astype(jnp.int32)]
    )
    fpos = jnp.take(nstart[:NG] - bounds[:NG], group_s) + idx
    perm = pks & ((1 << 18) - 1)
    sd_s = jnp.take(src * 16384 + dst, perm)
    src_s = sd_s >> 14
    dst_s = sd_s & 16383
    phase_s = group_s & (NPH - 1)
    srcl_s = src_s - jnp.minimum(phase_s * PW, N - PW)
    loc_s = dst_s - (group_s >> 2) * RPT
    packed = jnp.full((EPAD,), TRASH, jnp.int32).at[fpos].add(
        srcl_s * 1024 + loc_s - TRASH
    )
    src_pad = packed >> 10
    dl_pad = packed & 1023
    meta = jnp.zeros((NS, 16), jnp.int32)
    for p in range(NPH):
        g = jnp.arange(NS, dtype=jnp.int32) * NPH + p
        meta = meta.at[:, 2 * p].set(jnp.take(nstart, g))
        meta = meta.at[:, 2 * p + 1].set(jnp.take(nstart, g) + jnp.take(padded, g))
    return src_pad, dl_pad, meta.reshape(-1)


def _sc_segment_sum(feat, src_pad, dl_pad, meta):
    mesh = plsc.VectorSubcoreMesh(core_axis_name="c", subcore_axis_name="s")

    @functools.partial(
        pl.kernel,
        out_type=jax.ShapeDtypeStruct((NPAD, D), jnp.float32),
        mesh=mesh,
        scratch_types=[
            pltpu.VMEM_SHARED((PW, HD), jnp.float32),
            pltpu.VMEM((IB * K,), jnp.int32),
            pltpu.VMEM((IB * K,), jnp.int32),
            pltpu.VMEM((K, HD), jnp.float32),
            pltpu.VMEM((K, HD), jnp.float32),
            pltpu.VMEM((RPT + 1, HD), jnp.float32),
            pltpu.VMEM((16,), jnp.int32),
            pltpu.SemaphoreType.DMA,
            pltpu.SemaphoreType.DMA,
        ],
    )
    def k(feat_h, src_h, dl_h, meta_h, out_h, fsh, sblk, dblk, buf_a, buf_b, acc, st, sem_a, sem_b):
        c = lax.axis_index("c")
        s = lax.axis_index("s")

        zero = jnp.zeros((16,), jnp.float32)

        def zrow(r, _):
            for q in range(HD // 16):
                acc[r, pl.ds(q * 16, 16)] = zero
            return 0

        lax.fori_loop(0, RPT + 1, zrow, 0)

        pltpu.sync_copy(meta_h.at[pl.ds(pl.multiple_of(s * 16, 8), 16)], st)
        sv = st[...]

        for p in range(NPH):
            plsc.subcore_barrier()

            @pl.when(s == 0)
            def _():
                pltpu.sync_copy(
                    feat_h.at[pl.ds(min(p * PW, N - PW), PW), pl.ds(c * HD, HD)],
                    fsh,
                )

            plsc.subcore_barrier()
            s0 = sv[2 * p]
            s1 = sv[2 * p + 1]
            nch = (s1 - s0) // K
            nblk = (nch + IB - 1) // IB

            def blk_body(blk, _):
                base = pl.multiple_of(s0 + blk * (IB * K), 8)
                pltpu.sync_copy(src_h.at[pl.ds(base, IB * K)], sblk)
                pltpu.sync_copy(dl_h.at[pl.ds(base, IB * K)], dblk)
                nc_here = jnp.minimum(IB, nch - blk * IB)
                pltpu.async_copy(fsh.at[sblk.at[pl.ds(0, K)]], buf_a, sem_a)

                def accum(bref, doff):
                    def gbody(g, _):
                        dv = dblk[pl.ds(doff + g * 16, 16)]
                        for j in range(16):
                            d = dv[j]
                            row = g * 16 + j
                            for q in range(HD // 16):
                                plsc.addupdate(
                                    acc.at[d, pl.ds(q * 16, 16)],
                                    bref[row, pl.ds(q * 16, 16)],
                                )
                        return 0

                    lax.fori_loop(0, K // 16, gbody, 0)

                def pair(t, _):
                    c0 = t * 2
                    ob = pl.multiple_of((c0 + 1) * K, K)
                    cb = pltpu.async_copy(fsh.at[sblk.at[pl.ds(ob, K)]], buf_b, sem_b)
                    pltpu.make_async_copy(
                        feat_h.at[pl.ds(0, K), pl.ds(0, HD)], buf_a, sem_a
                    ).wait()
                    accum(buf_a, pl.multiple_of(c0 * K, K))

                    @pl.when(c0 + 2 < nc_here)
                    def _():
                        oa = pl.multiple_of((c0 + 2) * K, K)
                        pltpu.async_copy(fsh.at[sblk.at[pl.ds(oa, K)]], buf_a, sem_a)

                    cb.wait()
                    accum(buf_b, pl.multiple_of((c0 + 1) * K, K))
                    return 0

                lax.fori_loop(0, nc_here // 2, pair, 0)
                return 0

            lax.fori_loop(0, nblk, blk_body, 0)

        pltpu.sync_copy(
            acc.at[pl.ds(0, RPT)],
            out_h.at[pl.ds(s * RPT, RPT), pl.ds(c * HD, HD)],
        )

    return k(feat, src_pad, dl_pad, meta)


def _conv_tc(xin, agg, w_root, w_nbr, b):
    bm = 2000
    grid = N // bm

    def body(x_ref, a_ref, wr_ref, wn_ref, b_ref, o_ref):
        acc = jnp.dot(x_ref[...], wr_ref[...], preferred_element_type=jnp.float32)
        acc = acc + jnp.dot(a_ref[...], wn_ref[...], preferred_element_type=jnp.float32)
        o_ref[...] = jnp.maximum(acc + b_ref[...], 0.0)

    return pl.pallas_call(
        body,
        grid=(grid,),
        in_specs=[
            pl.BlockSpec((bm, D), lambda i: (i, 0)),
            pl.BlockSpec((bm, D), lambda i: (i, 0)),
            pl.BlockSpec((D, H), lambda i: (0, 0)),
            pl.BlockSpec((D, H), lambda i: (0, 0)),
            pl.BlockSpec((1, H), lambda i: (0, 0)),
        ],
        out_specs=pl.BlockSpec((bm, H), lambda i: (i, 0)),
        out_shape=jax.ShapeDtypeStruct((N, H), jnp.float32),
    )(xin, agg, w_root, w_nbr, b.reshape(1, H))


def _pool_tc(h, batch, lin_w, lin_b):
    bm = 2000
    grid = N // bm
    batch3 = batch.reshape(grid, 1, bm)

    def body(h_ref, b_ref, w_ref, bias_ref, o_ref, acc, cnt):
        i = pl.program_id(0)

        @pl.when(i == 0)
        def _():
            acc[...] = jnp.zeros_like(acc)
            cnt[...] = jnp.zeros_like(cnt)

        bb = b_ref[0, :, :]
        gids = lax.broadcasted_iota(jnp.int32, (G, bm), 0)
        oh = (gids == bb).astype(jnp.float32)
        acc[...] += jnp.dot(oh, h_ref[...], preferred_element_type=jnp.float32)
        cnt[...] += jnp.sum(oh, axis=1, keepdims=True)

        @pl.when(i == grid - 1)
        def _():
            pooled = acc[...] / jnp.maximum(cnt[...], 1.0)
            o_ref[...] = (
                jnp.dot(pooled, w_ref[...], preferred_element_type=jnp.float32)
                + bias_ref[...]
            )

    return pl.pallas_call(
        body,
        grid=(grid,),
        in_specs=[
            pl.BlockSpec((bm, H), lambda i: (i, 0)),
            pl.BlockSpec((1, 1, bm), lambda i: (i, 0, 0)),
            pl.BlockSpec((H, C), lambda i: (0, 0)),
            pl.BlockSpec((1, C), lambda i: (0, 0)),
        ],
        out_specs=pl.BlockSpec((G, C), lambda i: (0, 0)),
        out_shape=jax.ShapeDtypeStruct((G, C), jnp.float32),
        scratch_shapes=[
            pltpu.VMEM((G, H), jnp.float32),
            pltpu.VMEM((G, 1), jnp.float32),
        ],
    )(h, batch3, lin_w, lin_b.reshape(1, C))


def kernel(x, edge_index, batch, W1_root, W1_nbr, b1, W2_root, W2_nbr, b2, lin_W, lin_b):
    src_pad, dl_pad, meta = _prep_edges(edge_index[0], edge_index[1])
    agg1 = _sc_segment_sum(x, src_pad, dl_pad, meta)[:N]
    h1 = _conv_tc(x, agg1, W1_root, W1_nbr, b1)
    agg2 = _sc_segment_sum(h1, src_pad, dl_pad, meta)[:N]
    h2 = _conv_tc(h1, agg2, W2_root, W2_nbr, b2)
    return _pool_tc(h2, batch, lin_W, lin_b)


# speedup vs baseline: 2.4458x; 1.0120x over previous
import functools

import jax
import jax.numpy as jnp
from jax import lax
from jax.experimental import pallas as pl
from jax.experimental.pallas import tpu as pltpu
from jax.experimental.pallas import tpu_sc as plsc

N = 10000
E = 160000
D = 256
H = 256
C = 16
G = 64

NC = 2
NS = 16
HD = D // NC
K = 64
IB = 32
RPT = 640
TRASH = RPT
NPAD = NS * RPT
PW = 2560
NPH = 4
NG = NS * NPH
EPAD = E + NG * 2 * K + IB * K


def _prep_edges(src, dst):
    tile = dst // RPT
    phase = jnp.minimum(src // PW, NPH - 1)
    group = tile * NPH + phase
    idx = jnp.arange(E, dtype=jnp.int32)
    (pks,) = lax.sort([group * (1 << 18) + idx], is_stable=False)
    group_s = pks >> 18
    bounds = jnp.searchsorted(group_s, jnp.arange(NG + 1, dtype=jnp.int32)).astype(
        jnp.int32
    )
    cnt = bounds[1:] - bounds[:-1]
    padded = ((cnt + 2 * K - 1) // (2 * K)) * (2 * K)
    nstart = jnp.concatenate(
        [jnp.zeros((1,), jnp.int32), jnp.cumsum(padded).astype(jnp.int32)]
    )
    fpos = jnp.take(nstart[:NG] - bounds[:NG], group_s) + idx
    perm = pks & ((1 << 18) - 1)
    sd_s = jnp.take(src * 16384 + dst, perm)
    src_s = sd_s >> 14
    dst_s = sd_s & 16383
    phase_s = group_s & (NPH - 1)
    srcl_s = src_s - jnp.minimum(phase_s * PW, N - PW)
    loc_s = dst_s - (group_s >> 2) * RPT
    packed = jnp.full((EPAD,), TRASH, jnp.int32).at[fpos].add(
        srcl_s * 1024 + loc_s - TRASH
    )
    src_pad = packed >> 10
    dl_pad = packed & 1023
    meta = jnp.zeros((NS, 16), jnp.int32)
    for p in range(NPH):
        g = jnp.arange(NS, dtype=jnp.int32) * NPH + p
        meta = meta.at[:, 2 * p].set(jnp.take(nstart, g))
        meta = meta.at[:, 2 * p + 1].set(jnp.take(nstart, g) + jnp.take(padded, g))
    return src_pad, dl_pad, meta.reshape(-1)


def _sc_segment_sum(feat, src_pad, dl_pad, meta):
    mesh = plsc.VectorSubcoreMesh(core_axis_name="c", subcore_axis_name="s")

    @functools.partial(
        pl.kernel,
        out_type=jax.ShapeDtypeStruct((NPAD, D), jnp.float32),
        mesh=mesh,
        scratch_types=[
            pltpu.VMEM_SHARED((PW, HD), jnp.float32),
            pltpu.VMEM((IB * K,), jnp.int32),
            pltpu.VMEM((IB * K,), jnp.int32),
            pltpu.VMEM((K, HD), jnp.float32),
            pltpu.VMEM((K, HD), jnp.float32),
            pltpu.VMEM((RPT + 1, HD), jnp.float32),
            pltpu.VMEM((16,), jnp.int32),
            pltpu.SemaphoreType.DMA,
            pltpu.SemaphoreType.DMA,
        ],
    )
    def k(feat_h, src_h, dl_h, meta_h, out_h, fsh, sblk, dblk, buf_a, buf_b, acc, st, sem_a, sem_b):
        c = lax.axis_index("c")
        s = lax.axis_index("s")

        zero = jnp.zeros((16,), jnp.float32)

        def zrow(r, _):
            for q in range(HD // 16):
                acc[r, pl.ds(q * 16, 16)] = zero
            return 0

        lax.fori_loop(0, RPT + 1, zrow, 0)

        pltpu.sync_copy(meta_h.at[pl.ds(pl.multiple_of(s * 16, 8), 16)], st)
        sv = st[...]

        for p in range(NPH):
            plsc.subcore_barrier()

            @pl.when(s == 0)
            def _():
                pltpu.sync_copy(
                    feat_h.at[pl.ds(min(p * PW, N - PW), PW), pl.ds(c * HD, HD)],
                    fsh,
                )

            plsc.subcore_barrier()
            s0 = sv[2 * p]
            s1 = sv[2 * p + 1]
            nch = (s1 - s0) // K
            nblk = (nch + IB - 1) // IB

            def blk_body(blk, _):
                base = pl.multiple_of(s0 + blk * (IB * K), 8)
                pltpu.sync_copy(src_h.at[pl.ds(base, IB * K)], sblk)
                pltpu.sync_copy(dl_h.at[pl.ds(base, IB * K)], dblk)
                nc_here = jnp.minimum(IB, nch - blk * IB)
                pltpu.async_copy(fsh.at[sblk.at[pl.ds(0, K)]], buf_a, sem_a)

                def accum(bref, doff):
                    def gbody(g, _):
                        dv = dblk[pl.ds(doff + g * 16, 16)]
                        for j in range(16):
                            d = dv[j]
                            row = g * 16 + j
                            for q in range(HD // 16):
                                plsc.addupdate(
                                    acc.at[d, pl.ds(q * 16, 16)],
                                    bref[row, pl.ds(q * 16, 16)],
                                )
                        return 0

                    lax.fori_loop(0, K // 16, gbody, 0)

                def pair(t, _):
                    c0 = t * 2
                    ob = pl.multiple_of((c0 + 1) * K, K)
                    cb = pltpu.async_copy(fsh.at[sblk.at[pl.ds(ob, K)]], buf_b, sem_b)
                    pltpu.make_async_copy(
                        feat_h.at[pl.ds(0, K), pl.ds(0, HD)], buf_a, sem_a
                    ).wait()
                    accum(buf_a, pl.multiple_of(c0 * K, K))

                    @pl.when(c0 + 2 < nc_here)
                    def _():
                        oa = pl.multiple_of((c0 + 2) * K, K)
                        pltpu.async_copy(fsh.at[sblk.at[pl.ds(oa, K)]], buf_a, sem_a)

                    cb.wait()
                    accum(buf_b, pl.multiple_of((c0 + 1) * K, K))
                    return 0

                lax.fori_loop(0, nc_here // 2, pair, 0)
                return 0

            lax.fori_loop(0, nblk, blk_body, 0)

        pltpu.sync_copy(
            acc.at[pl.ds(0, RPT)],
            out_h.at[pl.ds(s * RPT, RPT), pl.ds(c * HD, HD)],
        )

    return k(feat, src_pad, dl_pad, meta)


def _conv_tc(xin, agg, w_root, w_nbr, b):
    bm = 2000
    grid = N // bm

    def body(x_ref, a_ref, wr_ref, wn_ref, b_ref, o_ref):
        acc = jnp.dot(x_ref[...], wr_ref[...], preferred_element_type=jnp.float32)
        acc = acc + jnp.dot(a_ref[...], wn_ref[...], preferred_element_type=jnp.float32)
        o_ref[...] = jnp.maximum(acc + b_ref[...], 0.0)

    return pl.pallas_call(
        body,
        grid=(grid,),
        in_specs=[
            pl.BlockSpec((bm, D), lambda i: (i, 0)),
            pl.BlockSpec((bm, D), lambda i: (i, 0)),
            pl.BlockSpec((D, H), lambda i: (0, 0)),
            pl.BlockSpec((D, H), lambda i: (0, 0)),
            pl.BlockSpec((1, H), lambda i: (0, 0)),
        ],
        out_specs=pl.BlockSpec((bm, H), lambda i: (i, 0)),
        out_shape=jax.ShapeDtypeStruct((N, H), jnp.float32),
    )(xin, agg, w_root, w_nbr, b.reshape(1, H))


def _pool_tc(h, batch, lin_w, lin_b):
    bm = 2000
    grid = N // bm
    batch3 = batch.reshape(grid, 1, bm)

    def body(h_ref, b_ref, w_ref, bias_ref, o_ref, acc, cnt):
        i = pl.program_id(0)

        @pl.when(i == 0)
        def _():
            acc[...] = jnp.zeros_like(acc)
            cnt[...] = jnp.zeros_like(cnt)

        bb = b_ref[0, :, :]
        gids = lax.broadcasted_iota(jnp.int32, (G, bm), 0)
        oh = (gids == bb).astype(jnp.float32)
        acc[...] += jnp.dot(oh, h_ref[...], preferred_element_type=jnp.float32)
        cnt[...] += jnp.sum(oh, axis=1, keepdims=True)

        @pl.when(i == grid - 1)
        def _():
            pooled = acc[...] / jnp.maximum(cnt[...], 1.0)
            o_ref[...] = (
                jnp.dot(pooled, w_ref[...], preferred_element_type=jnp.float32)
                + bias_ref[...]
            )

    return pl.pallas_call(
        body,
        grid=(grid,),
        in_specs=[
            pl.BlockSpec((bm, H), lambda i: (i, 0)),
            pl.BlockSpec((1, 1, bm), lambda i: (i, 0, 0)),
            pl.BlockSpec((H, C), lambda i: (0, 0)),
            pl.BlockSpec((1, C), lambda i: (0, 0)),
        ],
        out_specs=pl.BlockSpec((G, C), lambda i: (0, 0)),
        out_shape=jax.ShapeDtypeStruct((G, C), jnp.float32),
        scratch_shapes=[
            pltpu.VMEM((G, H), jnp.float32),
            pltpu.VMEM((G, 1), jnp.float32),
        ],
    )(h, batch3, lin_w, lin_b.reshape(1, C))


def kernel(x, edge_index, batch, W1_root, W1_nbr, b1, W2_root, W2_nbr, b2, lin_W, lin_b):
    src_pad, dl_pad, meta = _prep_edges(edge_index[0], edge_index[1])
    agg1 = _sc_segment_sum(x, src_pad, dl_pad, meta)[:N]
    h1 = _conv_tc(x, agg1, W1_root, W1_nbr, b1)
    agg2 = _sc_segment_sum(h1, src_pad, dl_pad, meta)[:N]
    h2 = _conv_tc(h1, agg2, W2_root, W2_nbr, b2)
    return _pool_tc(h2, batch, lin_W, lin_b)
